# baseline (speedup 1.0000x reference)
import functools

import jax
import jax.numpy as jnp
import numpy as np
from jax import lax
from jax.experimental import pallas as pl
from jax.experimental.pallas import tpu as pltpu
from jax.experimental.pallas import tpu_sc as plsc

_VSIZE = np.array([0.05, 0.05, 0.1], dtype=np.float32)
_PC_LO = np.array([0.0, -40.0, -3.0], dtype=np.float32)
_GX, _GY, _GZ = 1408, 1600, 40
_SENTINEL = _GX * _GY * _GZ
_MAX_VOXELS = 150000
_MAX_PTS = 5
_N = 300000
_NPAD = 300032
_ROWS0 = 2344

_NC, _NS = 2, 16
_CPTS = _NPAD // 32
_CV = _CPTS // 16
_HALF_VOX = _SENTINEL // 2
_PHALF = 45088768
_PROWS = _PHALF // 128
_G16H = _PHALF // 16
_G16_REAL = _HALF_VOX // 16
_PCROWS = _G16H // 128
_RANKS = 150016
_HRANK = _RANKS // 2
_HBLK = _HRANK // 128
_INVALID_VR = _RANKS
_VROWS_H = 375008
_VPAD = 375808
_VLPAD = 75264


def _k0_lin(x_ref, y_ref, z_ref, lin_ref):
    cx = jnp.floor((x_ref[...] - _PC_LO[0]) / _VSIZE[0]).astype(jnp.int32)
    cy = jnp.floor((y_ref[...] - _PC_LO[1]) / _VSIZE[1]).astype(jnp.int32)
    cz = jnp.floor((z_ref[...] - _PC_LO[2]) / _VSIZE[2]).astype(jnp.int32)
    valid = ((cx >= 0) & (cx < _GX) & (cy >= 0) & (cy < _GY)
             & (cz >= 0) & (cz < _GZ))
    lin = cz * (_GY * _GX) + cy * _GX + cx
    lin_ref[...] = jnp.where(valid, lin, _SENTINEL)


def _k1_body(lin_hbm, p0_hbm, p1_hbm, zbuf, linb, idxb, onesb, sem):
    cid = lax.axis_index("c")
    sid = lax.axis_index("s")
    iota = lax.iota(jnp.int32, 16)

    @pl.loop(0, 2048)
    def _zb(i):
        zbuf[pl.ds(i * 16, 16)] = jnp.zeros((16,), jnp.float32)

    @pl.loop(0, 586)
    def _ob(i):
        onesb[pl.ds(i * 16, 16)] = jnp.full((16,), 1.0, jnp.float32)

    def half(p_hbm, base):
        stripe = _PHALF // 16
        @pl.loop(0, stripe // 32768)
        def _z(i):
            pltpu.sync_copy(zbuf, p_hbm.at[pl.ds(sid * stripe + i * 32768,
                                                 32768)])
        plsc.subcore_barrier()
        for b in range(2):
            pbase = sid * (2 * _CPTS) + b * _CPTS
            pltpu.sync_copy(lin_hbm.at[pl.ds(pbase, _CPTS)], linb)

            @pl.loop(0, _CV)
            def _mk(k):
                l16 = linb[pl.ds(k * 16, 16)]
                inh = (l16 >= base) & (l16 < base + _HALF_VOX)
                dump = _HALF_VOX + sid * 64 + iota * 2
                idxb[pl.ds(k * 16, 16)] = jnp.where(inh, l16 - base, dump)
            pltpu.async_copy(onesb, p_hbm.at[idxb], sem).wait()

    @pl.when(cid == 0)
    def _():
        half(p0_hbm, 0)

    @pl.when(cid == 1)
    def _():
        half(p1_hbm, _HALF_VOX)


def _k2_pack(p_ref, w_ref, hv_ref, pc_ref):
    y = jnp.dot(p_ref[...], w_ref[...], preferred_element_type=jnp.float32)
    hv_ref[...] = y[:, :8].astype(jnp.int32)
    pc_ref[...] = y[:, 8:16]


def _k3_prefix(pc_ref, lt_ref, pref_ref, carry):
    i = pl.program_id(0)

    @pl.when(i == 0)
    def _():
        carry[0] = 0.0

    x = pc_ref[...]
    a = jnp.dot(x, lt_ref[...], preferred_element_type=jnp.float32)
    rs = a[:, 127:128]
    incl = rs
    for k in (1, 2, 4, 8, 16, 32, 64):
        incl = incl + jnp.concatenate(
            [jnp.zeros((k, 1), jnp.float32), incl[:-k]], axis=0)
    excl = incl - rs
    c0 = carry[0]
    pref_ref[...] = (a - x + excl + c0).astype(jnp.int32)
    carry[0] = c0 + jnp.sum(rs)


def _k4_body(lin_hbm, hv_hbm, pf_hbm, vr_hbm, sloc_hbm, tab_hbm,
             linb, idxb, hvb, pfb, vrb, tl, tbuf, sem):
    cid = lax.axis_index("c")
    sid = lax.axis_index("s")
    wid = sid * _NC + cid

    pltpu.sync_copy(lin_hbm.at[pl.ds(wid * _CPTS, _CPTS)], linb)

    @pl.loop(0, _CV)
    def _mk(k):
        l16 = linb[pl.ds(k * 16, 16)]
        g = l16 >> 4
        idxb[pl.ds(k * 16, 16)] = jnp.where(l16 >= _HALF_VOX, g + 2048, g)

    pltpu.async_copy(hv_hbm.at[idxb], hvb, sem).wait()
    pltpu.async_copy(pf_hbm.at[idxb], pfb, sem).wait()
    pltpu.sync_copy(pf_hbm.at[pl.ds(_G16_REAL, 16)], tbuf)
    t0 = tbuf[pl.ds(0, 16)][0]

    @pl.loop(0, _CV)
    def _rank(k):
        sl = pl.ds(k * 16, 16)
        l16 = linb[sl]
        m = hvb[sl]
        b = l16 & 15
        v = m & (jnp.left_shift(1, b) - 1)
        v = v - ((v >> 1) & 0x5555)
        v = (v & 0x3333) + ((v >> 2) & 0x3333)
        v = (v + (v >> 4)) & 0x0F0F
        pcnt = (v + (v >> 8)) & 0x1F
        rank = pfb[sl] + pcnt
        rank = jnp.where(l16 >= _HALF_VOX, rank + t0, rank)
        ok = (l16 < _SENTINEL) & (rank < _RANKS)
        vrb[sl] = jnp.where(ok, rank, _INVALID_VR)

    pltpu.sync_copy(vrb, vr_hbm.at[pl.ds(wid * _CPTS, _CPTS)])

    iota = lax.iota(jnp.int32, 16)
    one0 = jnp.where(iota == 0, 1, 0)
    zero16 = jnp.zeros((16,), jnp.int32)

    for hh in range(2):
        lo = hh * _HRANK

        @pl.loop(0, (_HRANK + 128) // 16)
        def _z(t):
            tl[pl.ds(t * 16, 16)] = jnp.zeros((16,), jnp.int32)

        @pl.loop(0, _CV)
        def _cnt(k):
            sl = pl.ds(k * 16, 16)
            v16 = vrb[sl]
            acc = jnp.zeros((16,), jnp.int32) if hh == 0 else idxb[sl]
            for l in range(16):
                a = v16[l]
                inh = (a >= lo) & (a < lo + _HRANK)
                inh_i = jnp.where(inh, 1, 0)
                addr = jnp.where(inh, a - lo, _HRANK)
                row = tl[pl.ds(addr, 16)]
                s = row[0]
                tl[pl.ds(addr, 16)] = row + one0 * inh_i
                sval = jnp.where(inh, s, acc[l])
                acc = jnp.where(iota == l, sval, acc)
            idxb[sl] = acc

        pltpu.sync_copy(tl.at[pl.ds(0, _HRANK)],
                        tab_hbm.at[wid, pl.ds(hh * _HRANK, _HRANK)])

    pltpu.sync_copy(idxb, sloc_hbm.at[pl.ds(wid * _CPTS, _CPTS)])


def _k5_scan(tab_ref, tabx_ref, nump_ref):
    x = tab_ref[...]
    incl = x
    for k in (1, 2, 4, 8, 16):
        incl = incl + jnp.concatenate(
            [jnp.zeros((k, 512), jnp.int32), incl[:-k, :]], axis=0)
    tabx_ref[...] = incl - x
    nump_ref[...] = jnp.minimum(incl[31:32, :], 5).reshape(1, 1, 512)


def _k6_body(px_hbm, py_hbm, pz_hbm, pw_hbm, lin_hbm, vr_hbm, sloc_hbm,
             tabx_hbm, zv_hbm, zi_hbm,
             v0_hbm, v1_hbm, vl0_hbm, vl1_hbm, pcid_hbm,
             zbufv, izbuf, xb, yb, zb, wb, linb, vrb, slocb, ib, sb, sem):
    cid = lax.axis_index("c")
    sid = lax.axis_index("s")
    iota = lax.iota(jnp.int32, 16)
    pltpu.sync_copy(zv_hbm, zbufv)
    pltpu.sync_copy(zi_hbm, izbuf)

    def half(v_hbm, vl_hbm, vrow_base, vrow_w, vl_base, vl_w, write_pcid):
        zf = _VPAD * 4 // 16
        off = sid * zf
        for nf in (16384, 16384, 16384, 16384, 16384, 12032):
            pltpu.sync_copy(zbufv.at[pl.ds(0, nf)],
                            v_hbm.at[pl.ds(off, nf)])
            off = off + nf
        pltpu.sync_copy(izbuf, vl_hbm.at[pl.ds(sid * 4704, 4704)])
        plsc.subcore_barrier()

        for b in range(2):
            c = 2 * sid + b
            pbase = c * _CPTS
            pltpu.sync_copy(px_hbm.at[pl.ds(pbase, _CPTS)], xb)
            pltpu.sync_copy(py_hbm.at[pl.ds(pbase, _CPTS)], yb)
            pltpu.sync_copy(pz_hbm.at[pl.ds(pbase, _CPTS)], zb)
            pltpu.sync_copy(pw_hbm.at[pl.ds(pbase, _CPTS)], wb)
            pltpu.sync_copy(lin_hbm.at[pl.ds(pbase, _CPTS)], linb)
            pltpu.sync_copy(vr_hbm.at[pl.ds(pbase, _CPTS)], vrb)
            pltpu.sync_copy(sloc_hbm.at[pl.ds(pbase, _CPTS)], slocb)

            @pl.loop(0, _CV)
            def _gi(k):
                vr16 = vrb[pl.ds(k * 16, 16)]
                ib[pl.ds(k * 16, 16)] = c * _RANKS + vr16
            pltpu.async_copy(tabx_hbm.at[ib], sb, sem).wait()

            for comp, src in ((0, xb), (1, yb), (2, zb), (3, wb)):
                @pl.loop(0, _CV)
                def _vx(k):
                    sl = pl.ds(k * 16, 16)
                    vr16 = vrb[sl]
                    slot = sb[sl] + slocb[sl]
                    stored = (vr16 < _MAX_VOXELS) & (slot < _MAX_PTS)
                    row = vr16 * _MAX_PTS + slot
                    inr = (stored & (row >= vrow_base)
                           & (row < vrow_base + vrow_w))
                    flat = (row - vrow_base) * 4 + comp
                    dump = _VROWS_H * 4 + sid * 64 + iota * 4 + comp
                    ib[sl] = jnp.where(inr, flat, dump)
                pltpu.async_copy(src, v_hbm.at[ib], sem).wait()

            @pl.loop(0, _CV)
            def _vl(k):
                sl = pl.ds(k * 16, 16)
                vr16 = vrb[sl]
                slot = sb[sl] + slocb[sl]
                stored = (vr16 < _MAX_VOXELS) & (slot < _MAX_PTS)
                inr = stored & (vr16 >= vl_base) & (vr16 < vl_base + vl_w)
                dump = _HRANK + sid * 16 + iota
                ib[sl] = jnp.where(inr, vr16 - vl_base, dump)
            pltpu.async_copy(linb, vl_hbm.at[ib], sem).wait()

            if write_pcid:
                @pl.loop(0, _CV)
                def _pc(k):
                    sl = pl.ds(k * 16, 16)
                    vr16 = vrb[sl]
                    slot = sb[sl] + slocb[sl]
                    stored = (vr16 < _MAX_VOXELS) & (slot < _MAX_PTS)
                    sb[sl] = jnp.where(stored, vr16, -1)
                pltpu.sync_copy(sb, pcid_hbm.at[pl.ds(pbase, _CPTS)])

    @pl.when(cid == 0)
    def _():
        half(v0_hbm, vl0_hbm, 0, _VROWS_H, 0, _HRANK, True)

    @pl.when(cid == 1)
    def _():
        half(v1_hbm, vl1_hbm, _VROWS_H, 750000 - _VROWS_H,
             _HRANK, _MAX_VOXELS - _HRANK, False)


def _k7_decode(vl_ref, cz_ref, cy_ref, cx_ref):
    vl = vl_ref[...]
    cz = vl // (_GY * _GX)
    rem = vl - cz * (_GY * _GX)
    cy = rem // _GX
    cz_ref[...] = cz.astype(jnp.float32)
    cy_ref[...] = cy.astype(jnp.float32)
    cx_ref[...] = (rem - cy * _GX).astype(jnp.float32)


_MESH = plsc.VectorSubcoreMesh(core_axis_name="c", subcore_axis_name="s")

_PACK_W = np.zeros((128, 16), np.float32)
for _l in range(128):
    _PACK_W[_l, _l >> 4] = float(1 << (_l & 15))
    _PACK_W[_l, 8 + (_l >> 4)] = 1.0
_LT = np.triu(np.ones((128, 128), np.float32))


def kernel(current_point):
    n = current_point.shape[0]
    f32, i32 = jnp.float32, jnp.int32
    pts_t = current_point.T
    pad = jnp.full((3, _NPAD - n), -1e9, f32)
    xyz = jnp.concatenate([pts_t[:3], pad], axis=1)
    lin2d = pl.pallas_call(
        _k0_lin,
        out_shape=jax.ShapeDtypeStruct((_ROWS0, 128), i32),
    )(xyz[0].reshape(_ROWS0, 128), xyz[1].reshape(_ROWS0, 128),
      xyz[2].reshape(_ROWS0, 128))
    lin = lin2d.reshape(_NPAD)

    p0, p1 = pl.kernel(
        _k1_body,
        out_type=(jax.ShapeDtypeStruct((_PHALF,), f32),
                  jax.ShapeDtypeStruct((_PHALF,), f32)),
        mesh=_MESH,
        scratch_types=(pltpu.VMEM((32768,), f32), pltpu.VMEM((_CPTS,), i32),
                       pltpu.VMEM((_CPTS,), i32), pltpu.VMEM((_CPTS,), f32),
                       pltpu.SemaphoreType.DMA),
    )(lin)

    w = jnp.asarray(_PACK_W)
    pack = pl.pallas_call(
        _k2_pack,
        grid=(172,),
        in_specs=[pl.BlockSpec((2048, 128), lambda i: (i, 0)),
                  pl.BlockSpec((128, 16), lambda i: (0, 0))],
        out_specs=[pl.BlockSpec((2048, 8), lambda i: (i, 0)),
                   pl.BlockSpec((2048, 8), lambda i: (i, 0))],
        out_shape=[jax.ShapeDtypeStruct((_PROWS, 8), i32),
                   jax.ShapeDtypeStruct((_PROWS, 8), f32)],
    )
    hv0, pc0 = pack(p0.reshape(_PROWS, 128), w)
    hv1, pc1 = pack(p1.reshape(_PROWS, 128), w)

    lt = jnp.asarray(_LT)
    prefix = pl.pallas_call(
        _k3_prefix,
        grid=(172,),
        in_specs=[pl.BlockSpec((128, 128), lambda i: (i, 0)),
                  pl.BlockSpec((128, 128), lambda i: (0, 0))],
        out_specs=[pl.BlockSpec((128, 128), lambda i: (i, 0))],
        out_shape=[jax.ShapeDtypeStruct((_PCROWS, 128), i32)],
        scratch_shapes=[pltpu.SMEM((1,), f32)],
    )
    pf0, = prefix(pc0.reshape(_PCROWS, 128), lt)
    pf1, = prefix(pc1.reshape(_PCROWS, 128), lt)

    hvcat = jnp.concatenate([hv0.reshape(-1), hv1.reshape(-1)])
    pfcat = jnp.concatenate([pf0.reshape(-1), pf1.reshape(-1)])

    vr, sloc, tab = pl.kernel(
        _k4_body,
        out_type=(jax.ShapeDtypeStruct((_NPAD,), i32),
                  jax.ShapeDtypeStruct((_NPAD,), i32),
                  jax.ShapeDtypeStruct((32, _RANKS), i32)),
        mesh=_MESH,
        scratch_types=(pltpu.VMEM((_CPTS,), i32), pltpu.VMEM((_CPTS,), i32),
                       pltpu.VMEM((_CPTS,), i32), pltpu.VMEM((_CPTS,), i32),
                       pltpu.VMEM((_CPTS,), i32),
                       pltpu.VMEM((_HRANK + 128,), i32),
                       pltpu.VMEM((16,), i32),
                       pltpu.SemaphoreType.DMA),
    )(lin, hvcat, pfcat)

    tabx, nump = pl.pallas_call(
        _k5_scan,
        grid=(293,),
        in_specs=[pl.BlockSpec((32, 512), lambda i: (0, i))],
        out_specs=[pl.BlockSpec((32, 512), lambda i: (0, i)),
                   pl.BlockSpec((1, 1, 512), lambda i: (i, 0, 0))],
        out_shape=[jax.ShapeDtypeStruct((32, _RANKS), i32),
                   jax.ShapeDtypeStruct((293, 1, 512), i32)],
    )(tab)
    tabxpad = jnp.concatenate([tabx.reshape(-1), jnp.zeros((128,), i32)])

    pts4 = jnp.concatenate(
        [pts_t, jnp.zeros((4, _NPAD - n), f32)], axis=1)
    v0b, v1b, vl0, vl1, pcid = pl.kernel(
        _k6_body,
        out_type=(jax.ShapeDtypeStruct((_VPAD * 4,), f32),
                  jax.ShapeDtypeStruct((_VPAD * 4,), f32),
                  jax.ShapeDtypeStruct((_VLPAD,), i32),
                  jax.ShapeDtypeStruct((_VLPAD,), i32),
                  jax.ShapeDtypeStruct((_NPAD,), i32)),
        mesh=_MESH,
        scratch_types=(pltpu.VMEM((16384,), f32), pltpu.VMEM((4704,), i32),
                       pltpu.VMEM((_CPTS,), f32), pltpu.VMEM((_CPTS,), f32),
                       pltpu.VMEM((_CPTS,), f32), pltpu.VMEM((_CPTS,), f32),
                       pltpu.VMEM((_CPTS,), i32), pltpu.VMEM((_CPTS,), i32),
                       pltpu.VMEM((_CPTS,), i32), pltpu.VMEM((_CPTS,), i32),
                       pltpu.VMEM((_CPTS,), i32),
                       pltpu.SemaphoreType.DMA),
    )(pts4[0], pts4[1], pts4[2], pts4[3], lin, vr, sloc, tabxpad,
      jnp.zeros((16384,), f32), jnp.zeros((4704,), i32))

    voxels = jnp.concatenate(
        [v0b[:_VROWS_H * 4], v1b[:(750000 - _VROWS_H) * 4]]).reshape(
            _MAX_VOXELS, _MAX_PTS, 4)
    vlcat = jnp.concatenate(
        [vl0[:_HRANK], vl1[:_MAX_VOXELS - _HRANK],
         jnp.zeros((150528 - _MAX_VOXELS,), i32)])
    cz, cy, cx = pl.pallas_call(
        _k7_decode,
        out_shape=[jax.ShapeDtypeStruct((1176, 128), f32)] * 3,
    )(vlcat.reshape(1176, 128))
    zcol = jnp.zeros((_MAX_VOXELS,), f32)
    voxel_coords = jnp.stack(
        [zcol, cz.reshape(-1)[:_MAX_VOXELS], cy.reshape(-1)[:_MAX_VOXELS],
         cx.reshape(-1)[:_MAX_VOXELS]], axis=1)

    voxel_num_points = nump.reshape(_RANKS)[:_MAX_VOXELS]
    pc_voxel_id = pcid[:n]
    return voxels, voxel_coords, voxel_num_points, pc_voxel_id


# speedup vs baseline: 6.4166x; 6.4166x over previous
import functools

import jax
import jax.numpy as jnp
import numpy as np
from jax import lax
from jax.experimental import pallas as pl
from jax.experimental.pallas import tpu as pltpu
from jax.experimental.pallas import tpu_sc as plsc

_VSIZE = np.array([0.05, 0.05, 0.1], dtype=np.float32)
_PC_LO = np.array([0.0, -40.0, -3.0], dtype=np.float32)
_GX, _GY, _GZ = 1408, 1600, 40
_SENTINEL = _GX * _GY * _GZ
_MAX_VOXELS = 150000
_MAX_PTS = 5
_N = 300000
_NPAD = 300032
_ROWS0 = 2344

_NC, _NS = 2, 16
_CPTS = _NPAD // 32
_CV = _CPTS // 16
_HALF_VOX = _SENTINEL // 2
_PHALF = 45088768
_PROWS = _PHALF // 128
_G16H = _PHALF // 16
_G16_REAL = _HALF_VOX // 16
_PCROWS = _G16H // 128
_RANKS = 150016
_HRANK = _RANKS // 2
_HBLK = _HRANK // 128
_INVALID_VR = _RANKS
_VROWS_H = 375008
_VPAD = 376832
_VLPAD = 75776
_SUBW = _VPAD


def _k0_lin(x_ref, y_ref, z_ref, lin_ref):
    cx = jnp.floor((x_ref[...] - _PC_LO[0]) / _VSIZE[0]).astype(jnp.int32)
    cy = jnp.floor((y_ref[...] - _PC_LO[1]) / _VSIZE[1]).astype(jnp.int32)
    cz = jnp.floor((z_ref[...] - _PC_LO[2]) / _VSIZE[2]).astype(jnp.int32)
    valid = ((cx >= 0) & (cx < _GX) & (cy >= 0) & (cy < _GY)
             & (cz >= 0) & (cz < _GZ))
    lin = cz * (_GY * _GX) + cy * _GX + cx
    lin_ref[...] = jnp.where(valid, lin, _SENTINEL)


def _k1_body(lin_hbm, p0_hbm, p1_hbm, zbuf, linb, idxb, onesb, sem):
    cid = lax.axis_index("c")
    sid = lax.axis_index("s")
    iota = lax.iota(jnp.int32, 16)

    @pl.loop(0, 2048)
    def _zb(i):
        zbuf[pl.ds(i * 16, 16)] = jnp.zeros((16,), jnp.float32)

    @pl.loop(0, 586)
    def _ob(i):
        onesb[pl.ds(i * 16, 16)] = jnp.full((16,), 1.0, jnp.float32)

    def half(p_hbm, base):
        stripe = _PHALF // 16
        @pl.loop(0, stripe // 32768)
        def _z(i):
            pltpu.sync_copy(zbuf, p_hbm.at[pl.ds(sid * stripe + i * 32768,
                                                 32768)])
        plsc.subcore_barrier()
        for b in range(2):
            pbase = sid * (2 * _CPTS) + b * _CPTS
            pltpu.sync_copy(lin_hbm.at[pl.ds(pbase, _CPTS)], linb)

            @pl.loop(0, _CV)
            def _mk(k):
                l16 = linb[pl.ds(k * 16, 16)]
                inh = (l16 >= base) & (l16 < base + _HALF_VOX)
                dump = _HALF_VOX + sid * 64 + iota * 2
                idxb[pl.ds(k * 16, 16)] = jnp.where(inh, l16 - base, dump)
            pltpu.async_copy(onesb, p_hbm.at[idxb], sem).wait()

    @pl.when(cid == 0)
    def _():
        half(p0_hbm, 0)

    @pl.when(cid == 1)
    def _():
        half(p1_hbm, _HALF_VOX)


def _k2_pack(p_ref, w_ref, hv_ref, pc_ref):
    y = jnp.dot(p_ref[...], w_ref[...], preferred_element_type=jnp.float32)
    hv_ref[...] = y[:, :8].astype(jnp.int32)
    pc_ref[...] = y[:, 8:16]


def _k3_prefix(pc_ref, lt_ref, pref_ref, carry):
    i = pl.program_id(0)

    @pl.when(i == 0)
    def _():
        carry[0] = 0.0

    x = pc_ref[...]
    a = jnp.dot(x, lt_ref[...], preferred_element_type=jnp.float32)
    rs = a[:, 127:128]
    incl = rs
    for k in (1, 2, 4, 8, 16, 32, 64):
        incl = incl + jnp.concatenate(
            [jnp.zeros((k, 1), jnp.float32), incl[:-k]], axis=0)
    excl = incl - rs
    c0 = carry[0]
    pref_ref[...] = (a - x + excl + c0).astype(jnp.int32)
    carry[0] = c0 + jnp.sum(rs)


def _k4_body(lin_hbm, hv_hbm, pf_hbm, vr_hbm, sloc_hbm, tab_hbm,
             linb, idxb, hvb, pfb, vrb, tl, tbuf, sem):
    cid = lax.axis_index("c")
    sid = lax.axis_index("s")
    wid = sid * _NC + cid

    pltpu.sync_copy(lin_hbm.at[pl.ds(wid * _CPTS, _CPTS)], linb)

    @pl.loop(0, _CV)
    def _mk(k):
        l16 = linb[pl.ds(k * 16, 16)]
        g = l16 >> 4
        idxb[pl.ds(k * 16, 16)] = jnp.where(l16 >= _HALF_VOX, g + 2048, g)

    pltpu.async_copy(hv_hbm.at[idxb], hvb, sem).wait()
    pltpu.async_copy(pf_hbm.at[idxb], pfb, sem).wait()
    pltpu.sync_copy(pf_hbm.at[pl.ds(_G16_REAL, 16)], tbuf)
    t0 = tbuf[pl.ds(0, 16)][0]

    @pl.loop(0, _CV)
    def _rank(k):
        sl = pl.ds(k * 16, 16)
        l16 = linb[sl]
        m = hvb[sl]
        b = l16 & 15
        v = m & (jnp.left_shift(1, b) - 1)
        v = v - ((v >> 1) & 0x5555)
        v = (v & 0x3333) + ((v >> 2) & 0x3333)
        v = (v + (v >> 4)) & 0x0F0F
        pcnt = (v + (v >> 8)) & 0x1F
        rank = pfb[sl] + pcnt
        rank = jnp.where(l16 >= _HALF_VOX, rank + t0, rank)
        ok = (l16 < _SENTINEL) & (rank < _RANKS)
        vrb[sl] = jnp.where(ok, rank, _INVALID_VR)

    pltpu.sync_copy(vrb, vr_hbm.at[pl.ds(wid * _CPTS, _CPTS)])

    iota = lax.iota(jnp.int32, 16)
    one0 = jnp.where(iota == 0, 1, 0)
    zero16 = jnp.zeros((16,), jnp.int32)

    for hh in range(2):
        lo = hh * _HRANK

        @pl.loop(0, (_HRANK + 128) // 16)
        def _z(t):
            tl[pl.ds(t * 16, 16)] = jnp.zeros((16,), jnp.int32)

        @pl.loop(0, _CV)
        def _cnt(k):
            sl = pl.ds(k * 16, 16)
            v16 = vrb[sl]
            acc = jnp.zeros((16,), jnp.int32) if hh == 0 else idxb[sl]
            for l in range(16):
                a = v16[l]
                inh = (a >= lo) & (a < lo + _HRANK)
                inh_i = jnp.where(inh, 1, 0)
                addr = jnp.where(inh, a - lo, _HRANK)
                row = tl[pl.ds(addr, 16)]
                s = row[0]
                tl[pl.ds(addr, 16)] = row + one0 * inh_i
                sval = jnp.where(inh, s, acc[l])
                acc = jnp.where(iota == l, sval, acc)
            idxb[sl] = acc

        pltpu.sync_copy(tl.at[pl.ds(0, _HRANK)],
                        tab_hbm.at[wid, pl.ds(hh * _HRANK, _HRANK)])

    pltpu.sync_copy(idxb, sloc_hbm.at[pl.ds(wid * _CPTS, _CPTS)])


def _k5_scan(tab_ref, tabx_ref, nump_ref):
    x = tab_ref[...]
    incl = x
    for k in (1, 2, 4, 8, 16):
        incl = incl + jnp.concatenate(
            [jnp.zeros((k, 512), jnp.int32), incl[:-k, :]], axis=0)
    tabx_ref[...] = incl - x
    nump_ref[...] = jnp.minimum(incl[31:32, :], 5).reshape(1, 1, 512)


def _k6_body(px_hbm, py_hbm, pz_hbm, pw_hbm, lin_hbm, vr_hbm, sloc_hbm,
             tabx_hbm,
             v0_hbm, v1_hbm, vl0_hbm, vl1_hbm, pcid_hbm,
             shv, shvl, zbufv, izbuf, xb, yb, zb, wb, linb, vrb, slocb,
             ib, sb, sem):
    cid = lax.axis_index("c")
    sid = lax.axis_index("s")
    iota = lax.iota(jnp.int32, 16)

    @pl.loop(0, 512)
    def _zf(i):
        zbufv[pl.ds(i * 16, 16)] = jnp.zeros((16,), jnp.float32)

    @pl.loop(0, 148)
    def _zi(i):
        izbuf[pl.ds(i * 16, 16)] = jnp.zeros((16,), jnp.int32)

    def half(v_hbm, vl_hbm, vrow_base, vrow_w, vl_base, vl_w, write_pcid):
        zf = _SUBW // 16
        for r in range(4):
            lo = r * _SUBW
            off = sid * zf
            for nf in (8192, 8192, 7168):
                pltpu.sync_copy(zbufv.at[pl.ds(0, nf)],
                                shv.at[pl.ds(off, nf)])
                off = off + nf
            if r == 0:
                pltpu.sync_copy(izbuf, shvl.at[pl.ds(sid * 4736, 2368)])
                pltpu.sync_copy(izbuf,
                                shvl.at[pl.ds(sid * 4736 + 2368, 2368)])
            plsc.subcore_barrier()

            for b in range(2):
                c = 2 * sid + b
                pbase = c * _CPTS
                pltpu.sync_copy(px_hbm.at[pl.ds(pbase, _CPTS)], xb)
                pltpu.sync_copy(py_hbm.at[pl.ds(pbase, _CPTS)], yb)
                pltpu.sync_copy(pz_hbm.at[pl.ds(pbase, _CPTS)], zb)
                pltpu.sync_copy(pw_hbm.at[pl.ds(pbase, _CPTS)], wb)
                pltpu.sync_copy(lin_hbm.at[pl.ds(pbase, _CPTS)], linb)
                pltpu.sync_copy(vr_hbm.at[pl.ds(pbase, _CPTS)], vrb)
                pltpu.sync_copy(sloc_hbm.at[pl.ds(pbase, _CPTS)], slocb)

                @pl.loop(0, _CV)
                def _gi(k):
                    vr16 = vrb[pl.ds(k * 16, 16)]
                    ib[pl.ds(k * 16, 16)] = c * _RANKS + vr16
                pltpu.async_copy(tabx_hbm.at[ib], sb, sem).wait()

                for comp, src in ((0, xb), (1, yb), (2, zb), (3, wb)):
                    @pl.loop(0, _CV)
                    def _vx(k):
                        sl = pl.ds(k * 16, 16)
                        vr16 = vrb[sl]
                        slot = sb[sl] + slocb[sl]
                        stored = (vr16 < _MAX_VOXELS) & (slot < _MAX_PTS)
                        row = vr16 * _MAX_PTS + slot
                        inr = (stored & (row >= vrow_base)
                               & (row < vrow_base + vrow_w))
                        flat = (row - vrow_base) * 4 + comp - lo
                        inr = inr & (flat >= 0) & (flat < _SUBW)
                        dump = _SUBW + sid * 64 + iota * 4 + comp
                        ib[sl] = jnp.where(inr, flat, dump)
                    pltpu.sync_copy(src, shv.at[ib])

                if r == 0:
                    @pl.loop(0, _CV)
                    def _vl(k):
                        sl = pl.ds(k * 16, 16)
                        vr16 = vrb[sl]
                        slot = sb[sl] + slocb[sl]
                        stored = (vr16 < _MAX_VOXELS) & (slot < _MAX_PTS)
                        inr = (stored & (vr16 >= vl_base)
                               & (vr16 < vl_base + vl_w))
                        dump = _HRANK + sid * 16 + iota
                        ib[sl] = jnp.where(inr, vr16 - vl_base, dump)
                    pltpu.sync_copy(linb, shvl.at[ib])

                if r == 0 and write_pcid:
                    @pl.loop(0, _CV)
                    def _pc(k):
                        sl = pl.ds(k * 16, 16)
                        vr16 = vrb[sl]
                        slot = sb[sl] + slocb[sl]
                        stored = (vr16 < _MAX_VOXELS) & (slot < _MAX_PTS)
                        sb[sl] = jnp.where(stored, vr16, -1)
                    pltpu.sync_copy(sb, pcid_hbm.at[pl.ds(pbase, _CPTS)])

            plsc.subcore_barrier()
            pltpu.sync_copy(shv.at[pl.ds(sid * zf, zf)],
                            v_hbm.at[pl.ds(lo + sid * zf, zf)])
            if r == 0:
                pltpu.sync_copy(shvl.at[pl.ds(sid * 4736, 4736)],
                                vl_hbm.at[pl.ds(sid * 4736, 4736)])
            plsc.subcore_barrier()

    @pl.when(cid == 0)
    def _():
        half(v0_hbm, vl0_hbm, 0, _VROWS_H, 0, _HRANK, True)

    @pl.when(cid == 1)
    def _():
        half(v1_hbm, vl1_hbm, _VROWS_H, 750000 - _VROWS_H,
             _HRANK, _MAX_VOXELS - _HRANK, False)


def _k7_decode(vl_ref, cz_ref, cy_ref, cx_ref):
    vl = vl_ref[...]
    cz = vl // (_GY * _GX)
    rem = vl - cz * (_GY * _GX)
    cy = rem // _GX
    cz_ref[...] = cz.astype(jnp.float32)
    cy_ref[...] = cy.astype(jnp.float32)
    cx_ref[...] = (rem - cy * _GX).astype(jnp.float32)


_MESH = plsc.VectorSubcoreMesh(core_axis_name="c", subcore_axis_name="s")

_PACK_W = np.zeros((128, 16), np.float32)
for _l in range(128):
    _PACK_W[_l, _l >> 4] = float(1 << (_l & 15))
    _PACK_W[_l, 8 + (_l >> 4)] = 1.0
_LT = np.triu(np.ones((128, 128), np.float32))


def kernel(current_point):
    n = current_point.shape[0]
    f32, i32 = jnp.float32, jnp.int32
    pts_t = current_point.T
    pad = jnp.full((3, _NPAD - n), -1e9, f32)
    xyz = jnp.concatenate([pts_t[:3], pad], axis=1)
    lin2d = pl.pallas_call(
        _k0_lin,
        out_shape=jax.ShapeDtypeStruct((_ROWS0, 128), i32),
    )(xyz[0].reshape(_ROWS0, 128), xyz[1].reshape(_ROWS0, 128),
      xyz[2].reshape(_ROWS0, 128))
    lin = lin2d.reshape(_NPAD)

    p0, p1 = pl.kernel(
        _k1_body,
        out_type=(jax.ShapeDtypeStruct((_PHALF,), f32),
                  jax.ShapeDtypeStruct((_PHALF,), f32)),
        mesh=_MESH,
        scratch_types=(pltpu.VMEM((32768,), f32), pltpu.VMEM((_CPTS,), i32),
                       pltpu.VMEM((_CPTS,), i32), pltpu.VMEM((_CPTS,), f32),
                       pltpu.SemaphoreType.DMA),
    )(lin)

    w = jnp.asarray(_PACK_W)
    pack = pl.pallas_call(
        _k2_pack,
        grid=(172,),
        in_specs=[pl.BlockSpec((2048, 128), lambda i: (i, 0)),
                  pl.BlockSpec((128, 16), lambda i: (0, 0))],
        out_specs=[pl.BlockSpec((2048, 8), lambda i: (i, 0)),
                   pl.BlockSpec((2048, 8), lambda i: (i, 0))],
        out_shape=[jax.ShapeDtypeStruct((_PROWS, 8), i32),
                   jax.ShapeDtypeStruct((_PROWS, 8), f32)],
    )
    hv0, pc0 = pack(p0.reshape(_PROWS, 128), w)
    hv1, pc1 = pack(p1.reshape(_PROWS, 128), w)

    lt = jnp.asarray(_LT)
    prefix = pl.pallas_call(
        _k3_prefix,
        grid=(172,),
        in_specs=[pl.BlockSpec((128, 128), lambda i: (i, 0)),
                  pl.BlockSpec((128, 128), lambda i: (0, 0))],
        out_specs=[pl.BlockSpec((128, 128), lambda i: (i, 0))],
        out_shape=[jax.ShapeDtypeStruct((_PCROWS, 128), i32)],
        scratch_shapes=[pltpu.SMEM((1,), f32)],
    )
    pf0, = prefix(pc0.reshape(_PCROWS, 128), lt)
    pf1, = prefix(pc1.reshape(_PCROWS, 128), lt)

    hvcat = jnp.concatenate([hv0.reshape(-1), hv1.reshape(-1)])
    pfcat = jnp.concatenate([pf0.reshape(-1), pf1.reshape(-1)])

    vr, sloc, tab = pl.kernel(
        _k4_body,
        out_type=(jax.ShapeDtypeStruct((_NPAD,), i32),
                  jax.ShapeDtypeStruct((_NPAD,), i32),
                  jax.ShapeDtypeStruct((32, _RANKS), i32)),
        mesh=_MESH,
        scratch_types=(pltpu.VMEM((_CPTS,), i32), pltpu.VMEM((_CPTS,), i32),
                       pltpu.VMEM((_CPTS,), i32), pltpu.VMEM((_CPTS,), i32),
                       pltpu.VMEM((_CPTS,), i32),
                       pltpu.VMEM((_HRANK + 128,), i32),
                       pltpu.VMEM((16,), i32),
                       pltpu.SemaphoreType.DMA),
    )(lin, hvcat, pfcat)

    tabx, nump = pl.pallas_call(
        _k5_scan,
        grid=(293,),
        in_specs=[pl.BlockSpec((32, 512), lambda i: (0, i))],
        out_specs=[pl.BlockSpec((32, 512), lambda i: (0, i)),
                   pl.BlockSpec((1, 1, 512), lambda i: (i, 0, 0))],
        out_shape=[jax.ShapeDtypeStruct((32, _RANKS), i32),
                   jax.ShapeDtypeStruct((293, 1, 512), i32)],
    )(tab)
    tabxpad = jnp.concatenate([tabx.reshape(-1), jnp.zeros((128,), i32)])

    pts4 = jnp.concatenate(
        [pts_t, jnp.zeros((4, _NPAD - n), f32)], axis=1)
    v0b, v1b, vl0, vl1, pcid = pl.kernel(
        _k6_body,
        out_type=(jax.ShapeDtypeStruct((_VPAD * 4,), f32),
                  jax.ShapeDtypeStruct((_VPAD * 4,), f32),
                  jax.ShapeDtypeStruct((_VLPAD,), i32),
                  jax.ShapeDtypeStruct((_VLPAD,), i32),
                  jax.ShapeDtypeStruct((_NPAD,), i32)),
        mesh=_MESH,
        scratch_types=(pltpu.VMEM_SHARED((_SUBW + 1024,), f32),
                       pltpu.VMEM_SHARED((_VLPAD,), i32),
                       pltpu.VMEM((8192,), f32), pltpu.VMEM((2368,), i32),
                       pltpu.VMEM((_CPTS,), f32), pltpu.VMEM((_CPTS,), f32),
                       pltpu.VMEM((_CPTS,), f32), pltpu.VMEM((_CPTS,), f32),
                       pltpu.VMEM((_CPTS,), i32), pltpu.VMEM((_CPTS,), i32),
                       pltpu.VMEM((_CPTS,), i32), pltpu.VMEM((_CPTS,), i32),
                       pltpu.VMEM((_CPTS,), i32),
                       pltpu.SemaphoreType.DMA),
    )(pts4[0], pts4[1], pts4[2], pts4[3], lin, vr, sloc, tabxpad)

    voxels = jnp.concatenate(
        [v0b[:_VROWS_H * 4], v1b[:(750000 - _VROWS_H) * 4]]).reshape(
            _MAX_VOXELS, _MAX_PTS, 4)
    vlcat = jnp.concatenate(
        [vl0[:_HRANK], vl1[:_MAX_VOXELS - _HRANK],
         jnp.zeros((150528 - _MAX_VOXELS,), i32)])
    cz, cy, cx = pl.pallas_call(
        _k7_decode,
        out_shape=[jax.ShapeDtypeStruct((1176, 128), f32)] * 3,
    )(vlcat.reshape(1176, 128))
    zcol = jnp.zeros((_MAX_VOXELS,), f32)
    voxel_coords = jnp.stack(
        [zcol, cz.reshape(-1)[:_MAX_VOXELS], cy.reshape(-1)[:_MAX_VOXELS],
         cx.reshape(-1)[:_MAX_VOXELS]], axis=1)

    voxel_num_points = nump.reshape(_RANKS)[:_MAX_VOXELS]
    pc_voxel_id = pcid[:n]
    return voxels, voxel_coords, voxel_num_points, pc_voxel_id


# speedup vs baseline: 9.5948x; 1.4953x over previous
import functools

import jax
import jax.numpy as jnp
import numpy as np
from jax import lax
from jax.experimental import pallas as pl
from jax.experimental.pallas import tpu as pltpu
from jax.experimental.pallas import tpu_sc as plsc

_VSIZE = np.array([0.05, 0.05, 0.1], dtype=np.float32)
_PC_LO = np.array([0.0, -40.0, -3.0], dtype=np.float32)
_GX, _GY, _GZ = 1408, 1600, 40
_SENTINEL = _GX * _GY * _GZ
_MAX_VOXELS = 150000
_MAX_PTS = 5
_N = 300000
_NPAD = 300032
_ROWS0 = 2344

_NC, _NS = 2, 16
_CPTS = _NPAD // 32
_CV = _CPTS // 16
_HALF_VOX = _SENTINEL // 2
_PHALF = 45088768
_PROWS = _PHALF // 128
_G16H = _PHALF // 16
_G16_REAL = _HALF_VOX // 16
_PCROWS = _G16H // 128
_RANKS = 150016
_HRANK = _RANKS // 2
_HBLK = _HRANK // 128
_INVALID_VR = _RANKS
_VROWS_H = 375008
_VPAD = 376832
_VLPAD = 75776
_SUBW = _VPAD


def _k0_lin(x_ref, y_ref, z_ref, lin_ref):
    cx = jnp.floor((x_ref[...] - _PC_LO[0]) / _VSIZE[0]).astype(jnp.int32)
    cy = jnp.floor((y_ref[...] - _PC_LO[1]) / _VSIZE[1]).astype(jnp.int32)
    cz = jnp.floor((z_ref[...] - _PC_LO[2]) / _VSIZE[2]).astype(jnp.int32)
    valid = ((cx >= 0) & (cx < _GX) & (cy >= 0) & (cy < _GY)
             & (cz >= 0) & (cz < _GZ))
    lin = cz * (_GY * _GX) + cy * _GX + cx
    lin_ref[...] = jnp.where(valid, lin, _SENTINEL)


def _k1_body(lin_hbm, p0_hbm, p1_hbm, zbuf, linb, idxb, onesb, sem):
    cid = lax.axis_index("c")
    sid = lax.axis_index("s")
    iota = lax.iota(jnp.int32, 16)

    @pl.loop(0, 2048)
    def _zb(i):
        zbuf[pl.ds(i * 16, 16)] = jnp.zeros((16,), jnp.float32)

    @pl.loop(0, 586)
    def _ob(i):
        onesb[pl.ds(i * 16, 16)] = jnp.full((16,), 1.0, jnp.float32)

    def half(p_hbm, base):
        stripe = _PHALF // 16
        @pl.loop(0, stripe // 32768)
        def _z(i):
            pltpu.sync_copy(zbuf, p_hbm.at[pl.ds(sid * stripe + i * 32768,
                                                 32768)])
        plsc.subcore_barrier()
        for b in range(2):
            pbase = sid * (2 * _CPTS) + b * _CPTS
            pltpu.sync_copy(lin_hbm.at[pl.ds(pbase, _CPTS)], linb)

            @pl.loop(0, _CV)
            def _mk(k):
                l16 = linb[pl.ds(k * 16, 16)]
                inh = (l16 >= base) & (l16 < base + _HALF_VOX)
                dump = (_HALF_VOX + sid * 2048 + (k & 127) * 16 + iota)
                idxb[pl.ds(k * 16, 16)] = jnp.where(inh, l16 - base, dump)
            pltpu.async_copy(onesb, p_hbm.at[idxb], sem).wait()

    @pl.when(cid == 0)
    def _():
        half(p0_hbm, 0)

    @pl.when(cid == 1)
    def _():
        half(p1_hbm, _HALF_VOX)


def _k2_pack(p_ref, w_ref, hv_ref, pc_ref):
    y = jnp.dot(p_ref[...], w_ref[...], preferred_element_type=jnp.float32)
    hv_ref[...] = y[:, :8].astype(jnp.int32)
    pc_ref[...] = y[:, 8:16]


def _k3_prefix(pc_ref, lt_ref, pref_ref, carry):
    i = pl.program_id(0)

    @pl.when(i == 0)
    def _():
        carry[0] = 0.0

    x = pc_ref[...]
    a = jnp.dot(x, lt_ref[...], preferred_element_type=jnp.float32)
    rs = a[:, 127:128]
    incl = rs
    for k in (1, 2, 4, 8, 16, 32, 64):
        incl = incl + jnp.concatenate(
            [jnp.zeros((k, 1), jnp.float32), incl[:-k]], axis=0)
    excl = incl - rs
    c0 = carry[0]
    pref_ref[...] = (a - x + excl + c0).astype(jnp.int32)
    carry[0] = c0 + jnp.sum(rs)


def _k4_body(lin_hbm, hv_hbm, pf_hbm, vr_hbm, sloc_hbm, tab_hbm,
             linb, idxb, hvb, pfb, vrb, tl, tbuf, sem):
    cid = lax.axis_index("c")
    sid = lax.axis_index("s")
    wid = sid * _NC + cid

    pltpu.sync_copy(lin_hbm.at[pl.ds(wid * _CPTS, _CPTS)], linb)

    @pl.loop(0, _CV)
    def _mk(k):
        l16 = linb[pl.ds(k * 16, 16)]
        g = l16 >> 4
        idxb[pl.ds(k * 16, 16)] = jnp.where(l16 >= _HALF_VOX, g + 2048, g)

    pltpu.async_copy(hv_hbm.at[idxb], hvb, sem).wait()
    pltpu.async_copy(pf_hbm.at[idxb], pfb, sem).wait()
    pltpu.sync_copy(pf_hbm.at[pl.ds(_G16_REAL, 16)], tbuf)
    t0 = tbuf[pl.ds(0, 16)][0]

    @pl.loop(0, _CV)
    def _rank(k):
        sl = pl.ds(k * 16, 16)
        l16 = linb[sl]
        m = hvb[sl]
        b = l16 & 15
        v = m & (jnp.left_shift(1, b) - 1)
        v = v - ((v >> 1) & 0x5555)
        v = (v & 0x3333) + ((v >> 2) & 0x3333)
        v = (v + (v >> 4)) & 0x0F0F
        pcnt = (v + (v >> 8)) & 0x1F
        rank = pfb[sl] + pcnt
        rank = jnp.where(l16 >= _HALF_VOX, rank + t0, rank)
        ok = (l16 < _SENTINEL) & (rank < _RANKS)
        vrb[sl] = jnp.where(ok, rank, _INVALID_VR)

    pltpu.sync_copy(vrb, vr_hbm.at[pl.ds(wid * _CPTS, _CPTS)])

    iota = lax.iota(jnp.int32, 16)
    one0 = jnp.where(iota == 0, 1, 0)
    zero16 = jnp.zeros((16,), jnp.int32)

    for hh in range(2):
        lo = hh * _HRANK

        @pl.loop(0, (_HRANK + 128) // 16)
        def _z(t):
            tl[pl.ds(t * 16, 16)] = jnp.zeros((16,), jnp.int32)

        @pl.loop(0, _CV)
        def _cnt(k):
            sl = pl.ds(k * 16, 16)
            v16 = vrb[sl]
            acc = jnp.zeros((16,), jnp.int32) if hh == 0 else idxb[sl]
            for l in range(16):
                a = v16[l]
                inh = (a >= lo) & (a < lo + _HRANK)
                inh_i = jnp.where(inh, 1, 0)
                addr = jnp.where(inh, a - lo, _HRANK)
                row = tl[pl.ds(addr, 16)]
                s = row[0]
                tl[pl.ds(addr, 16)] = row + one0 * inh_i
                sval = jnp.where(inh, s, acc[l])
                acc = jnp.where(iota == l, sval, acc)
            idxb[sl] = acc

        pltpu.sync_copy(tl.at[pl.ds(0, _HRANK)],
                        tab_hbm.at[wid, pl.ds(hh * _HRANK, _HRANK)])

    pltpu.sync_copy(idxb, sloc_hbm.at[pl.ds(wid * _CPTS, _CPTS)])


def _k5_scan(tab_ref, tabx_ref, nump_ref):
    x = tab_ref[...]
    incl = x
    for k in (1, 2, 4, 8, 16):
        incl = incl + jnp.concatenate(
            [jnp.zeros((k, 512), jnp.int32), incl[:-k, :]], axis=0)
    tabx_ref[...] = incl - x
    nump_ref[...] = jnp.minimum(incl[31:32, :], 5).reshape(1, 1, 512)


def _k6_body(px_hbm, py_hbm, pz_hbm, pw_hbm, lin_hbm, vr_hbm, sloc_hbm,
             tabx_hbm,
             v0_hbm, v1_hbm, vl0_hbm, vl1_hbm, pcid_hbm,
             shv, shvl, zbufv, izbuf, xb, yb, zb, wb, linb, vrb, slocb,
             ib, sb, sem):
    cid = lax.axis_index("c")
    sid = lax.axis_index("s")
    iota = lax.iota(jnp.int32, 16)

    @pl.loop(0, 512)
    def _zf(i):
        zbufv[pl.ds(i * 16, 16)] = jnp.zeros((16,), jnp.float32)

    @pl.loop(0, 148)
    def _zi(i):
        izbuf[pl.ds(i * 16, 16)] = jnp.zeros((16,), jnp.int32)

    def half(v_hbm, vl_hbm, vrow_base, vrow_w, vl_base, vl_w, write_pcid):
        zf = _SUBW // 16
        for r in range(4):
            lo = r * _SUBW
            off = sid * zf
            for nf in (8192, 8192, 7168):
                pltpu.sync_copy(zbufv.at[pl.ds(0, nf)],
                                shv.at[pl.ds(off, nf)])
                off = off + nf
            if r == 0:
                pltpu.sync_copy(izbuf, shvl.at[pl.ds(sid * 4736, 2368)])
                pltpu.sync_copy(izbuf,
                                shvl.at[pl.ds(sid * 4736 + 2368, 2368)])
            plsc.subcore_barrier()

            for b in range(2):
                c = 2 * sid + b
                pbase = c * _CPTS
                pltpu.sync_copy(px_hbm.at[pl.ds(pbase, _CPTS)], xb)
                pltpu.sync_copy(py_hbm.at[pl.ds(pbase, _CPTS)], yb)
                pltpu.sync_copy(pz_hbm.at[pl.ds(pbase, _CPTS)], zb)
                pltpu.sync_copy(pw_hbm.at[pl.ds(pbase, _CPTS)], wb)
                pltpu.sync_copy(lin_hbm.at[pl.ds(pbase, _CPTS)], linb)
                pltpu.sync_copy(vr_hbm.at[pl.ds(pbase, _CPTS)], vrb)
                pltpu.sync_copy(sloc_hbm.at[pl.ds(pbase, _CPTS)], slocb)

                @pl.loop(0, _CV)
                def _gi(k):
                    vr16 = vrb[pl.ds(k * 16, 16)]
                    ib[pl.ds(k * 16, 16)] = c * _RANKS + vr16
                pltpu.async_copy(tabx_hbm.at[ib], sb, sem).wait()

                for comp, src in ((0, xb), (1, yb), (2, zb), (3, wb)):
                    @pl.loop(0, _CV)
                    def _vx(k):
                        sl = pl.ds(k * 16, 16)
                        vr16 = vrb[sl]
                        slot = sb[sl] + slocb[sl]
                        stored = (vr16 < _MAX_VOXELS) & (slot < _MAX_PTS)
                        row = vr16 * _MAX_PTS + slot
                        inr = (stored & (row >= vrow_base)
                               & (row < vrow_base + vrow_w))
                        flat = (row - vrow_base) * 4 + comp - lo
                        inr = inr & (flat >= 0) & (flat < _SUBW)
                        dump = _SUBW + sid * 64 + iota * 4 + comp
                        ib[sl] = jnp.where(inr, flat, dump)
                    pltpu.sync_copy(src, shv.at[ib])

                if r == 0:
                    @pl.loop(0, _CV)
                    def _vl(k):
                        sl = pl.ds(k * 16, 16)
                        vr16 = vrb[sl]
                        slot = sb[sl] + slocb[sl]
                        stored = (vr16 < _MAX_VOXELS) & (slot < _MAX_PTS)
                        inr = (stored & (vr16 >= vl_base)
                               & (vr16 < vl_base + vl_w))
                        dump = _HRANK + sid * 16 + iota
                        ib[sl] = jnp.where(inr, vr16 - vl_base, dump)
                    pltpu.sync_copy(linb, shvl.at[ib])

                if r == 0 and write_pcid:
                    @pl.loop(0, _CV)
                    def _pc(k):
                        sl = pl.ds(k * 16, 16)
                        vr16 = vrb[sl]
                        slot = sb[sl] + slocb[sl]
                        stored = (vr16 < _MAX_VOXELS) & (slot < _MAX_PTS)
                        sb[sl] = jnp.where(stored, vr16, -1)
                    pltpu.sync_copy(sb, pcid_hbm.at[pl.ds(pbase, _CPTS)])

            plsc.subcore_barrier()
            pltpu.sync_copy(shv.at[pl.ds(sid * zf, zf)],
                            v_hbm.at[pl.ds(lo + sid * zf, zf)])
            if r == 0:
                pltpu.sync_copy(shvl.at[pl.ds(sid * 4736, 4736)],
                                vl_hbm.at[pl.ds(sid * 4736, 4736)])
            plsc.subcore_barrier()

    @pl.when(cid == 0)
    def _():
        half(v0_hbm, vl0_hbm, 0, _VROWS_H, 0, _HRANK, True)

    @pl.when(cid == 1)
    def _():
        half(v1_hbm, vl1_hbm, _VROWS_H, 750000 - _VROWS_H,
             _HRANK, _MAX_VOXELS - _HRANK, False)


def _k7_decode(vl_ref, cz_ref, cy_ref, cx_ref):
    vl = vl_ref[...]
    cz = vl // (_GY * _GX)
    rem = vl - cz * (_GY * _GX)
    cy = rem // _GX
    cz_ref[...] = cz.astype(jnp.float32)
    cy_ref[...] = cy.astype(jnp.float32)
    cx_ref[...] = (rem - cy * _GX).astype(jnp.float32)


_MESH = plsc.VectorSubcoreMesh(core_axis_name="c", subcore_axis_name="s")

_PACK_W = np.zeros((128, 16), np.float32)
for _l in range(128):
    _PACK_W[_l, _l >> 4] = float(1 << (_l & 15))
    _PACK_W[_l, 8 + (_l >> 4)] = 1.0
_LT = np.triu(np.ones((128, 128), np.float32))


def kernel(current_point):
    n = current_point.shape[0]
    f32, i32 = jnp.float32, jnp.int32
    pts_t = current_point.T
    pad = jnp.full((3, _NPAD - n), -1e9, f32)
    xyz = jnp.concatenate([pts_t[:3], pad], axis=1)
    lin2d = pl.pallas_call(
        _k0_lin,
        out_shape=jax.ShapeDtypeStruct((_ROWS0, 128), i32),
    )(xyz[0].reshape(_ROWS0, 128), xyz[1].reshape(_ROWS0, 128),
      xyz[2].reshape(_ROWS0, 128))
    lin = lin2d.reshape(_NPAD)

    p0, p1 = pl.kernel(
        _k1_body,
        out_type=(jax.ShapeDtypeStruct((_PHALF,), f32),
                  jax.ShapeDtypeStruct((_PHALF,), f32)),
        mesh=_MESH,
        scratch_types=(pltpu.VMEM((32768,), f32), pltpu.VMEM((_CPTS,), i32),
                       pltpu.VMEM((_CPTS,), i32), pltpu.VMEM((_CPTS,), f32),
                       pltpu.SemaphoreType.DMA),
    )(lin)

    w = jnp.asarray(_PACK_W)
    pack = pl.pallas_call(
        _k2_pack,
        grid=(172,),
        in_specs=[pl.BlockSpec((2048, 128), lambda i: (i, 0)),
                  pl.BlockSpec((128, 16), lambda i: (0, 0))],
        out_specs=[pl.BlockSpec((2048, 8), lambda i: (i, 0)),
                   pl.BlockSpec((2048, 8), lambda i: (i, 0))],
        out_shape=[jax.ShapeDtypeStruct((_PROWS, 8), i32),
                   jax.ShapeDtypeStruct((_PROWS, 8), f32)],
    )
    hv0, pc0 = pack(p0.reshape(_PROWS, 128), w)
    hv1, pc1 = pack(p1.reshape(_PROWS, 128), w)

    lt = jnp.asarray(_LT)
    prefix = pl.pallas_call(
        _k3_prefix,
        grid=(172,),
        in_specs=[pl.BlockSpec((128, 128), lambda i: (i, 0)),
                  pl.BlockSpec((128, 128), lambda i: (0, 0))],
        out_specs=[pl.BlockSpec((128, 128), lambda i: (i, 0))],
        out_shape=[jax.ShapeDtypeStruct((_PCROWS, 128), i32)],
        scratch_shapes=[pltpu.SMEM((1,), f32)],
    )
    pf0, = prefix(pc0.reshape(_PCROWS, 128), lt)
    pf1, = prefix(pc1.reshape(_PCROWS, 128), lt)

    hvcat = jnp.concatenate([hv0.reshape(-1), hv1.reshape(-1)])
    pfcat = jnp.concatenate([pf0.reshape(-1), pf1.reshape(-1)])

    vr, sloc, tab = pl.kernel(
        _k4_body,
        out_type=(jax.ShapeDtypeStruct((_NPAD,), i32),
                  jax.ShapeDtypeStruct((_NPAD,), i32),
                  jax.ShapeDtypeStruct((32, _RANKS), i32)),
        mesh=_MESH,
        scratch_types=(pltpu.VMEM((_CPTS,), i32), pltpu.VMEM((_CPTS,), i32),
                       pltpu.VMEM((_CPTS,), i32), pltpu.VMEM((_CPTS,), i32),
                       pltpu.VMEM((_CPTS,), i32),
                       pltpu.VMEM((_HRANK + 128,), i32),
                       pltpu.VMEM((16,), i32),
                       pltpu.SemaphoreType.DMA),
    )(lin, hvcat, pfcat)

    tabx, nump = pl.pallas_call(
        _k5_scan,
        grid=(293,),
        in_specs=[pl.BlockSpec((32, 512), lambda i: (0, i))],
        out_specs=[pl.BlockSpec((32, 512), lambda i: (0, i)),
                   pl.BlockSpec((1, 1, 512), lambda i: (i, 0, 0))],
        out_shape=[jax.ShapeDtypeStruct((32, _RANKS), i32),
                   jax.ShapeDtypeStruct((293, 1, 512), i32)],
    )(tab)
    tabxpad = jnp.concatenate([tabx.reshape(-1), jnp.zeros((128,), i32)])

    pts4 = jnp.concatenate(
        [pts_t, jnp.zeros((4, _NPAD - n), f32)], axis=1)
    v0b, v1b, vl0, vl1, pcid = pl.kernel(
        _k6_body,
        out_type=(jax.ShapeDtypeStruct((_VPAD * 4,), f32),
                  jax.ShapeDtypeStruct((_VPAD * 4,), f32),
                  jax.ShapeDtypeStruct((_VLPAD,), i32),
                  jax.ShapeDtypeStruct((_VLPAD,), i32),
                  jax.ShapeDtypeStruct((_NPAD,), i32)),
        mesh=_MESH,
        scratch_types=(pltpu.VMEM_SHARED((_SUBW + 1024,), f32),
                       pltpu.VMEM_SHARED((_VLPAD,), i32),
                       pltpu.VMEM((8192,), f32), pltpu.VMEM((2368,), i32),
                       pltpu.VMEM((_CPTS,), f32), pltpu.VMEM((_CPTS,), f32),
                       pltpu.VMEM((_CPTS,), f32), pltpu.VMEM((_CPTS,), f32),
                       pltpu.VMEM((_CPTS,), i32), pltpu.VMEM((_CPTS,), i32),
                       pltpu.VMEM((_CPTS,), i32), pltpu.VMEM((_CPTS,), i32),
                       pltpu.VMEM((_CPTS,), i32),
                       pltpu.SemaphoreType.DMA),
    )(pts4[0], pts4[1], pts4[2], pts4[3], lin, vr, sloc, tabxpad)

    voxels = jnp.concatenate(
        [v0b[:_VROWS_H * 4], v1b[:(750000 - _VROWS_H) * 4]]).reshape(
            _MAX_VOXELS, _MAX_PTS, 4)
    vlcat = jnp.concatenate(
        [vl0[:_HRANK], vl1[:_MAX_VOXELS - _HRANK],
         jnp.zeros((150528 - _MAX_VOXELS,), i32)])
    cz, cy, cx = pl.pallas_call(
        _k7_decode,
        out_shape=[jax.ShapeDtypeStruct((1176, 128), f32)] * 3,
    )(vlcat.reshape(1176, 128))
    zcol = jnp.zeros((_MAX_VOXELS,), f32)
    voxel_coords = jnp.stack(
        [zcol, cz.reshape(-1)[:_MAX_VOXELS], cy.reshape(-1)[:_MAX_VOXELS],
         cx.reshape(-1)[:_MAX_VOXELS]], axis=1)

    voxel_num_points = nump.reshape(_RANKS)[:_MAX_VOXELS]
    pc_voxel_id = pcid[:n]
    return voxels, voxel_coords, voxel_num_points, pc_voxel_id


# speedup vs baseline: 10.5756x; 1.1022x over previous
import functools

import jax
import jax.numpy as jnp
import numpy as np
from jax import lax
from jax.experimental import pallas as pl
from jax.experimental.pallas import tpu as pltpu
from jax.experimental.pallas import tpu_sc as plsc

_VSIZE = np.array([0.05, 0.05, 0.1], dtype=np.float32)
_PC_LO = np.array([0.0, -40.0, -3.0], dtype=np.float32)
_GX, _GY, _GZ = 1408, 1600, 40
_SENTINEL = _GX * _GY * _GZ
_MAX_VOXELS = 150000
_MAX_PTS = 5
_N = 300000
_NPAD = 300032
_ROWS0 = 2344

_NC, _NS = 2, 16
_CPTS = _NPAD // 32
_CV = _CPTS // 16
_HALF_VOX = _SENTINEL // 2
_PHALF = 45088768
_PROWS = _PHALF // 128
_G16H = _PHALF // 16
_G16_REAL = _HALF_VOX // 16
_PCROWS = _G16H // 128
_RANKS = 150016
_HRANK = _RANKS // 2
_HBLK = _HRANK // 128
_INVALID_VR = _RANKS
_VROWS_H = 375008
_VPAD = 376832
_VLPAD = 75776
_SUBW = _VPAD


def _k0_lin(x_ref, y_ref, z_ref, lin_ref):
    cx = jnp.floor((x_ref[...] - _PC_LO[0]) / _VSIZE[0]).astype(jnp.int32)
    cy = jnp.floor((y_ref[...] - _PC_LO[1]) / _VSIZE[1]).astype(jnp.int32)
    cz = jnp.floor((z_ref[...] - _PC_LO[2]) / _VSIZE[2]).astype(jnp.int32)
    valid = ((cx >= 0) & (cx < _GX) & (cy >= 0) & (cy < _GY)
             & (cz >= 0) & (cz < _GZ))
    lin = cz * (_GY * _GX) + cy * _GX + cx
    lin_ref[...] = jnp.where(valid, lin, _SENTINEL)


def _k1_body(lin_hbm, p0_hbm, p1_hbm, zbuf, linb, idxb, onesb, sem):
    cid = lax.axis_index("c")
    sid = lax.axis_index("s")
    iota = lax.iota(jnp.int32, 16)

    @pl.loop(0, 2048)
    def _zb(i):
        zbuf[pl.ds(i * 16, 16)] = jnp.zeros((16,), jnp.float32)

    @pl.loop(0, 586)
    def _ob(i):
        onesb[pl.ds(i * 16, 16)] = jnp.full((16,), 1.0, jnp.float32)

    def half(p_hbm, base):
        stripe = _PHALF // 16
        @pl.loop(0, stripe // 32768)
        def _z(i):
            pltpu.sync_copy(zbuf, p_hbm.at[pl.ds(sid * stripe + i * 32768,
                                                 32768)])
        plsc.subcore_barrier()
        for b in range(2):
            pbase = sid * (2 * _CPTS) + b * _CPTS
            pltpu.sync_copy(lin_hbm.at[pl.ds(pbase, _CPTS)], linb)

            @pl.loop(0, _CV)
            def _mk(k):
                l16 = linb[pl.ds(k * 16, 16)]
                inh = (l16 >= base) & (l16 < base + _HALF_VOX)
                dump = (_HALF_VOX + sid * 2048 + (k & 127) * 16 + iota)
                idxb[pl.ds(k * 16, 16)] = jnp.where(inh, l16 - base, dump)
            pltpu.async_copy(onesb, p_hbm.at[idxb], sem).wait()

    @pl.when(cid == 0)
    def _():
        half(p0_hbm, 0)

    @pl.when(cid == 1)
    def _():
        half(p1_hbm, _HALF_VOX)


def _k2_pack(p_ref, w_ref, hv_ref, pc_ref):
    y = jnp.dot(p_ref[...], w_ref[...], preferred_element_type=jnp.float32)
    hv_ref[...] = y[:, :8].astype(jnp.int32)
    pc_ref[...] = y[:, 8:16]


def _k3_prefix(pc_ref, lt_ref, pref_ref, carry):
    i = pl.program_id(0)

    @pl.when(i == 0)
    def _():
        carry[0] = 0.0

    x = pc_ref[...]
    a = jnp.dot(x, lt_ref[...], preferred_element_type=jnp.float32)
    rs = a[:, 127:128]
    incl = rs
    for k in (1, 2, 4, 8, 16, 32, 64, 128, 256):
        incl = incl + jnp.concatenate(
            [jnp.zeros((k, 1), jnp.float32), incl[:-k]], axis=0)
    excl = incl - rs
    c0 = carry[0]
    pref_ref[...] = (a - x + excl + c0).astype(jnp.int32)
    carry[0] = c0 + jnp.sum(rs)


def _k4_body(lin_hbm, hv_hbm, pf_hbm, vr_hbm, sloc_hbm, tab_hbm,
             linb, idxb, hvb, pfb, vrb, tl, tbuf, sem):
    cid = lax.axis_index("c")
    sid = lax.axis_index("s")
    wid = sid * _NC + cid

    pltpu.sync_copy(lin_hbm.at[pl.ds(wid * _CPTS, _CPTS)], linb)

    @pl.loop(0, _CV)
    def _mk(k):
        l16 = linb[pl.ds(k * 16, 16)]
        g = l16 >> 4
        idxb[pl.ds(k * 16, 16)] = jnp.where(l16 >= _HALF_VOX, g + 2048, g)

    pltpu.async_copy(hv_hbm.at[idxb], hvb, sem).wait()
    pltpu.async_copy(pf_hbm.at[idxb], pfb, sem).wait()
    pltpu.sync_copy(pf_hbm.at[pl.ds(_G16_REAL, 16)], tbuf)
    t0 = tbuf[pl.ds(0, 16)][0]

    @pl.loop(0, _CV)
    def _rank(k):
        sl = pl.ds(k * 16, 16)
        l16 = linb[sl]
        m = hvb[sl]
        b = l16 & 15
        v = m & (jnp.left_shift(1, b) - 1)
        v = v - ((v >> 1) & 0x5555)
        v = (v & 0x3333) + ((v >> 2) & 0x3333)
        v = (v + (v >> 4)) & 0x0F0F
        pcnt = (v + (v >> 8)) & 0x1F
        rank = pfb[sl] + pcnt
        rank = jnp.where(l16 >= _HALF_VOX, rank + t0, rank)
        ok = (l16 < _SENTINEL) & (rank < _RANKS)
        vrb[sl] = jnp.where(ok, rank, _INVALID_VR)

    pltpu.sync_copy(vrb, vr_hbm.at[pl.ds(wid * _CPTS, _CPTS)])

    iota = lax.iota(jnp.int32, 16)
    one0 = jnp.where(iota == 0, 1, 0)
    zero16 = jnp.zeros((16,), jnp.int32)

    for hh in range(2):
        lo = hh * _HRANK

        @pl.loop(0, (_HRANK + 128) // 16)
        def _z(t):
            tl[pl.ds(t * 16, 16)] = jnp.zeros((16,), jnp.int32)

        @pl.loop(0, _CV)
        def _cnt(k):
            sl = pl.ds(k * 16, 16)
            v16 = vrb[sl]
            acc = jnp.zeros((16,), jnp.int32) if hh == 0 else idxb[sl]
            for l in range(16):
                a = v16[l]
                inh = (a >= lo) & (a < lo + _HRANK)
                inh_i = jnp.where(inh, 1, 0)
                addr = jnp.where(inh, a - lo, _HRANK)
                row = tl[pl.ds(addr, 16)]
                s = row[0]
                tl[pl.ds(addr, 16)] = row + one0 * inh_i
                sval = jnp.where(inh, s, acc[l])
                acc = jnp.where(iota == l, sval, acc)
            idxb[sl] = acc

        pltpu.sync_copy(tl.at[pl.ds(0, _HRANK)],
                        tab_hbm.at[wid, pl.ds(hh * _HRANK, _HRANK)])

    pltpu.sync_copy(idxb, sloc_hbm.at[pl.ds(wid * _CPTS, _CPTS)])


def _k5_scan(tab_ref, tabx_ref, nump_ref):
    x = tab_ref[...]
    incl = x
    for k in (1, 2, 4, 8, 16):
        incl = incl + jnp.concatenate(
            [jnp.zeros((k, 512), jnp.int32), incl[:-k, :]], axis=0)
    tabx_ref[...] = incl - x
    nump_ref[...] = jnp.minimum(incl[31:32, :], 5).reshape(1, 1, 512)


def _k6_body(px_hbm, py_hbm, pz_hbm, pw_hbm, lin_hbm, vr_hbm, sloc_hbm,
             tabx_hbm,
             v0_hbm, v1_hbm, vl0_hbm, vl1_hbm, pcid_hbm,
             shv, shvl, zbufv, izbuf, xb, yb, zb, wb, linb, vrb, slocb,
             ib, sb, sem):
    cid = lax.axis_index("c")
    sid = lax.axis_index("s")
    iota = lax.iota(jnp.int32, 16)

    @pl.loop(0, 512)
    def _zf(i):
        zbufv[pl.ds(i * 16, 16)] = jnp.zeros((16,), jnp.float32)

    @pl.loop(0, 148)
    def _zi(i):
        izbuf[pl.ds(i * 16, 16)] = jnp.zeros((16,), jnp.int32)

    def half(v_hbm, vl_hbm, vrow_base, vrow_w, vl_base, vl_w, write_pcid):
        zf = _SUBW // 16
        for r in range(4):
            lo = r * _SUBW
            off = sid * zf
            for nf in (8192, 8192, 7168):
                pltpu.sync_copy(zbufv.at[pl.ds(0, nf)],
                                shv.at[pl.ds(off, nf)])
                off = off + nf
            if r == 0:
                pltpu.sync_copy(izbuf, shvl.at[pl.ds(sid * 4736, 2368)])
                pltpu.sync_copy(izbuf,
                                shvl.at[pl.ds(sid * 4736 + 2368, 2368)])
            plsc.subcore_barrier()

            for b in range(2):
                c = 2 * sid + b
                pbase = c * _CPTS
                pltpu.sync_copy(px_hbm.at[pl.ds(pbase, _CPTS)], xb)
                pltpu.sync_copy(py_hbm.at[pl.ds(pbase, _CPTS)], yb)
                pltpu.sync_copy(pz_hbm.at[pl.ds(pbase, _CPTS)], zb)
                pltpu.sync_copy(pw_hbm.at[pl.ds(pbase, _CPTS)], wb)
                pltpu.sync_copy(lin_hbm.at[pl.ds(pbase, _CPTS)], linb)
                pltpu.sync_copy(vr_hbm.at[pl.ds(pbase, _CPTS)], vrb)
                pltpu.sync_copy(sloc_hbm.at[pl.ds(pbase, _CPTS)], slocb)

                @pl.loop(0, _CV)
                def _gi(k):
                    vr16 = vrb[pl.ds(k * 16, 16)]
                    ib[pl.ds(k * 16, 16)] = c * _RANKS + vr16
                pltpu.async_copy(tabx_hbm.at[ib], sb, sem).wait()

                for comp, src in ((0, xb), (1, yb), (2, zb), (3, wb)):
                    @pl.loop(0, _CV)
                    def _vx(k):
                        sl = pl.ds(k * 16, 16)
                        vr16 = vrb[sl]
                        slot = sb[sl] + slocb[sl]
                        stored = (vr16 < _MAX_VOXELS) & (slot < _MAX_PTS)
                        row = vr16 * _MAX_PTS + slot
                        inr = (stored & (row >= vrow_base)
                               & (row < vrow_base + vrow_w))
                        flat = (row - vrow_base) * 4 + comp - lo
                        inr = inr & (flat >= 0) & (flat < _SUBW)
                        dump = _SUBW + sid * 64 + iota * 4 + comp
                        ib[sl] = jnp.where(inr, flat, dump)
                    pltpu.sync_copy(src, shv.at[ib])

                if r == 0:
                    @pl.loop(0, _CV)
                    def _vl(k):
                        sl = pl.ds(k * 16, 16)
                        vr16 = vrb[sl]
                        slot = sb[sl] + slocb[sl]
                        stored = (vr16 < _MAX_VOXELS) & (slot < _MAX_PTS)
                        inr = (stored & (vr16 >= vl_base)
                               & (vr16 < vl_base + vl_w))
                        dump = _HRANK + sid * 16 + iota
                        ib[sl] = jnp.where(inr, vr16 - vl_base, dump)
                    pltpu.sync_copy(linb, shvl.at[ib])

                if r == 0 and write_pcid:
                    @pl.loop(0, _CV)
                    def _pc(k):
                        sl = pl.ds(k * 16, 16)
                        vr16 = vrb[sl]
                        slot = sb[sl] + slocb[sl]
                        stored = (vr16 < _MAX_VOXELS) & (slot < _MAX_PTS)
                        sb[sl] = jnp.where(stored, vr16, -1)
                    pltpu.sync_copy(sb, pcid_hbm.at[pl.ds(pbase, _CPTS)])

            plsc.subcore_barrier()
            pltpu.sync_copy(shv.at[pl.ds(sid * zf, zf)],
                            v_hbm.at[pl.ds(lo + sid * zf, zf)])
            if r == 0:
                pltpu.sync_copy(shvl.at[pl.ds(sid * 4736, 4736)],
                                vl_hbm.at[pl.ds(sid * 4736, 4736)])
            plsc.subcore_barrier()

    @pl.when(cid == 0)
    def _():
        half(v0_hbm, vl0_hbm, 0, _VROWS_H, 0, _HRANK, True)

    @pl.when(cid == 1)
    def _():
        half(v1_hbm, vl1_hbm, _VROWS_H, 750000 - _VROWS_H,
             _HRANK, _MAX_VOXELS - _HRANK, False)


def _k7_decode(vl_ref, cz_ref, cy_ref, cx_ref):
    vl = vl_ref[...]
    cz = vl // (_GY * _GX)
    rem = vl - cz * (_GY * _GX)
    cy = rem // _GX
    cz_ref[...] = cz.astype(jnp.float32)
    cy_ref[...] = cy.astype(jnp.float32)
    cx_ref[...] = (rem - cy * _GX).astype(jnp.float32)


_MESH = plsc.VectorSubcoreMesh(core_axis_name="c", subcore_axis_name="s")

_PACK_W = np.zeros((128, 16), np.float32)
for _l in range(128):
    _PACK_W[_l, _l >> 4] = float(1 << (_l & 15))
    _PACK_W[_l, 8 + (_l >> 4)] = 1.0
_LT = np.triu(np.ones((128, 128), np.float32))


def kernel(current_point):
    n = current_point.shape[0]
    f32, i32 = jnp.float32, jnp.int32
    pts_t = current_point.T
    pad = jnp.full((3, _NPAD - n), -1e9, f32)
    xyz = jnp.concatenate([pts_t[:3], pad], axis=1)
    lin2d = pl.pallas_call(
        _k0_lin,
        out_shape=jax.ShapeDtypeStruct((_ROWS0, 128), i32),
    )(xyz[0].reshape(_ROWS0, 128), xyz[1].reshape(_ROWS0, 128),
      xyz[2].reshape(_ROWS0, 128))
    lin = lin2d.reshape(_NPAD)

    p0, p1 = pl.kernel(
        _k1_body,
        out_type=(jax.ShapeDtypeStruct((_PHALF,), f32),
                  jax.ShapeDtypeStruct((_PHALF,), f32)),
        mesh=_MESH,
        scratch_types=(pltpu.VMEM((32768,), f32), pltpu.VMEM((_CPTS,), i32),
                       pltpu.VMEM((_CPTS,), i32), pltpu.VMEM((_CPTS,), f32),
                       pltpu.SemaphoreType.DMA),
    )(lin)

    w = jnp.asarray(_PACK_W)
    pack = pl.pallas_call(
        _k2_pack,
        grid=(43,),
        in_specs=[pl.BlockSpec((8192, 128), lambda i: (i, 0)),
                  pl.BlockSpec((128, 16), lambda i: (0, 0))],
        out_specs=[pl.BlockSpec((8192, 8), lambda i: (i, 0)),
                   pl.BlockSpec((8192, 8), lambda i: (i, 0))],
        out_shape=[jax.ShapeDtypeStruct((_PROWS, 8), i32),
                   jax.ShapeDtypeStruct((_PROWS, 8), f32)],
    )
    hv0, pc0 = pack(p0.reshape(_PROWS, 128), w)
    hv1, pc1 = pack(p1.reshape(_PROWS, 128), w)

    lt = jnp.asarray(_LT)
    prefix = pl.pallas_call(
        _k3_prefix,
        grid=(43,),
        in_specs=[pl.BlockSpec((512, 128), lambda i: (i, 0)),
                  pl.BlockSpec((128, 128), lambda i: (0, 0))],
        out_specs=[pl.BlockSpec((512, 128), lambda i: (i, 0))],
        out_shape=[jax.ShapeDtypeStruct((_PCROWS, 128), i32)],
        scratch_shapes=[pltpu.SMEM((1,), f32)],
    )
    pf0, = prefix(pc0.reshape(_PCROWS, 128), lt)
    pf1, = prefix(pc1.reshape(_PCROWS, 128), lt)

    hvcat = jnp.concatenate([hv0.reshape(-1), hv1.reshape(-1)])
    pfcat = jnp.concatenate([pf0.reshape(-1), pf1.reshape(-1)])

    vr, sloc, tab = pl.kernel(
        _k4_body,
        out_type=(jax.ShapeDtypeStruct((_NPAD,), i32),
                  jax.ShapeDtypeStruct((_NPAD,), i32),
                  jax.ShapeDtypeStruct((32, _RANKS), i32)),
        mesh=_MESH,
        scratch_types=(pltpu.VMEM((_CPTS,), i32), pltpu.VMEM((_CPTS,), i32),
                       pltpu.VMEM((_CPTS,), i32), pltpu.VMEM((_CPTS,), i32),
                       pltpu.VMEM((_CPTS,), i32),
                       pltpu.VMEM((_HRANK + 128,), i32),
                       pltpu.VMEM((16,), i32),
                       pltpu.SemaphoreType.DMA),
    )(lin, hvcat, pfcat)

    tabx, nump = pl.pallas_call(
        _k5_scan,
        grid=(293,),
        in_specs=[pl.BlockSpec((32, 512), lambda i: (0, i))],
        out_specs=[pl.BlockSpec((32, 512), lambda i: (0, i)),
                   pl.BlockSpec((1, 1, 512), lambda i: (i, 0, 0))],
        out_shape=[jax.ShapeDtypeStruct((32, _RANKS), i32),
                   jax.ShapeDtypeStruct((293, 1, 512), i32)],
    )(tab)
    tabxpad = jnp.concatenate([tabx.reshape(-1), jnp.zeros((128,), i32)])

    pts4 = jnp.concatenate(
        [pts_t, jnp.zeros((4, _NPAD - n), f32)], axis=1)
    v0b, v1b, vl0, vl1, pcid = pl.kernel(
        _k6_body,
        out_type=(jax.ShapeDtypeStruct((_VPAD * 4,), f32),
                  jax.ShapeDtypeStruct((_VPAD * 4,), f32),
                  jax.ShapeDtypeStruct((_VLPAD,), i32),
                  jax.ShapeDtypeStruct((_VLPAD,), i32),
                  jax.ShapeDtypeStruct((_NPAD,), i32)),
        mesh=_MESH,
        scratch_types=(pltpu.VMEM_SHARED((_SUBW + 1024,), f32),
                       pltpu.VMEM_SHARED((_VLPAD,), i32),
                       pltpu.VMEM((8192,), f32), pltpu.VMEM((2368,), i32),
                       pltpu.VMEM((_CPTS,), f32), pltpu.VMEM((_CPTS,), f32),
                       pltpu.VMEM((_CPTS,), f32), pltpu.VMEM((_CPTS,), f32),
                       pltpu.VMEM((_CPTS,), i32), pltpu.VMEM((_CPTS,), i32),
                       pltpu.VMEM((_CPTS,), i32), pltpu.VMEM((_CPTS,), i32),
                       pltpu.VMEM((_CPTS,), i32),
                       pltpu.SemaphoreType.DMA),
    )(pts4[0], pts4[1], pts4[2], pts4[3], lin, vr, sloc, tabxpad)

    voxels = jnp.concatenate(
        [v0b[:_VROWS_H * 4], v1b[:(750000 - _VROWS_H) * 4]]).reshape(
            _MAX_VOXELS, _MAX_PTS, 4)
    vlcat = jnp.concatenate(
        [vl0[:_HRANK], vl1[:_MAX_VOXELS - _HRANK],
         jnp.zeros((150528 - _MAX_VOXELS,), i32)])
    cz, cy, cx = pl.pallas_call(
        _k7_decode,
        out_shape=[jax.ShapeDtypeStruct((1176, 128), f32)] * 3,
    )(vlcat.reshape(1176, 128))
    zcol = jnp.zeros((_MAX_VOXELS,), f32)
    voxel_coords = jnp.stack(
        [zcol, cz.reshape(-1)[:_MAX_VOXELS], cy.reshape(-1)[:_MAX_VOXELS],
         cx.reshape(-1)[:_MAX_VOXELS]], axis=1)

    voxel_num_points = nump.reshape(_RANKS)[:_MAX_VOXELS]
    pc_voxel_id = pcid[:n]
    return voxels, voxel_coords, voxel_num_points, pc_voxel_id
